# baseline (speedup 1.0000x reference)
import functools

import jax
import jax.numpy as jnp
from jax import lax
from jax.experimental import pallas as pl
from jax.experimental.pallas import tpu as pltpu
from jax.experimental.pallas import tpu_sc as plsc

_NC = 2
_NS = 16
_NW = _NC * _NS
_L = 16
_K = 32
_CH = 26


def _sc_scatter_body(M, B, rpw, rpw_pad, idx_hbm, val_hbm, in_hbm, out_ref,
                     idx_v, lastpos_v, selb_v, seldst_v, cbuf_v,
                     sem_c, sem_g, sem_s):
    wid = lax.axis_index("s") * _NC + lax.axis_index("c")
    base = wid * rpw
    span_main = M - (_NW - 1) * rpw
    tail = span_main % _L

    pltpu.sync_copy(idx_hbm, idx_v)

    minus1 = jnp.full((_L,), -1, jnp.int32)

    def init_body(i, _):
        lastpos_v[pl.ds(i * _L, _L)] = minus1
        return 0

    lax.fori_loop(0, rpw_pad // _L, init_body, 0, unroll=4)

    iota = lax.iota(jnp.int32, _L)

    def scan_body(i, _):
        v = idx_v[pl.ds(i * _L, _L)]
        owned = (v >= base) & (v < base + rpw)
        _, lastmask = plsc.scan_count(v, owned)
        keep = lastmask & owned
        bvec = iota + i * _L
        plsc.store_scatter(lastpos_v, [v - base], bvec, mask=keep)
        return 0

    lax.fori_loop(0, B // _L, scan_body, 0, unroll=4)

    def compact_body(i, off):
        lp = lastpos_v[pl.ds(i * _L, _L)]
        m = lp >= 0
        plsc.store_compressed(selb_v.at[pl.ds(off, _L)], lp, mask=m)
        plsc.store_compressed(
            seldst_v.at[pl.ds(off, _L)], iota + (base + i * _L), mask=m)
        return off + jnp.sum(m.astype(jnp.int32))

    cnt = lax.fori_loop(0, rpw_pad // _L, compact_body, 0, unroll=4)

    span = jnp.where(wid == _NW - 1, span_main, rpw)
    nfull = span // _CH
    ntail = span - nfull * _CH

    def _cgather(c, slot):
        return pltpu.make_async_copy(
            in_hbm.at[pl.ds(base + c * _CH, _CH)], cbuf_v.at[slot], sem_c)

    def _cscatter(c, slot):
        return pltpu.make_async_copy(
            cbuf_v.at[slot], out_ref.at[pl.ds(base + c * _CH, _CH)], sem_s)

    _cgather(0, 0).start()

    def copy_body(c, _):
        slot = c % 2

        @pl.when(c > 0)
        def _wait_prev_store():
            _cscatter(c - 1, 1 - slot).wait()

        @pl.when(c + 1 < nfull)
        def _fire_next_load():
            _cgather(c + 1, 1 - slot).start()

        _cgather(c, slot).wait()
        _cscatter(c, slot).start()
        return 0

    lax.fori_loop(0, nfull, copy_body, 0)
    _cscatter(nfull - 1, (nfull - 1) % 2).wait()

    toff = base + nfull * _CH
    for j in range(_CH - 1):
        @pl.when(j < ntail)
        def _tg():
            pltpu.async_copy(in_hbm.at[toff + j], cbuf_v.at[0, j], sem_c)
    for j in range(_CH - 1):
        @pl.when(j < ntail)
        def _tgw():
            pltpu.make_async_copy(
                in_hbm.at[toff + j], cbuf_v.at[0, j], sem_c).wait()
    for j in range(_CH - 1):
        @pl.when(j < ntail)
        def _ts():
            pltpu.async_copy(cbuf_v.at[0, j], out_ref.at[toff + j], sem_s)
    for j in range(_CH - 1):
        @pl.when(j < ntail)
        def _tsw():
            pltpu.make_async_copy(
                cbuf_v.at[0, j], out_ref.at[toff + j], sem_s).wait()

    @pl.when(cnt > 0)
    def _move():
        def chunk_body(c, _):
            o = c * _L
            bv = selb_v[pl.ds(o, _L)]
            dv = seldst_v[pl.ds(o, _L)]
            slot = c % 2
            for j in range(_L):
                @pl.when(o + j < cnt)
                def _fire_gather():
                    pltpu.async_copy(
                        val_hbm.at[bv[j]], cbuf_v.at[slot, j], sem_g)

            @pl.when(c > 0)
            def _drain_prev_scatters():
                po = (c - 1) * _L
                pbv = selb_v[pl.ds(po, _L)]
                pdv = seldst_v[pl.ds(po, _L)]
                for j in range(_L):
                    @pl.when(po + j < cnt)
                    def _drain_s():
                        pltpu.make_async_copy(
                            cbuf_v.at[1 - slot, j], out_ref.at[pdv[j]],
                            sem_s).wait()

            for j in range(_L):
                @pl.when(o + j < cnt)
                def _drain_g():
                    pltpu.make_async_copy(
                        val_hbm.at[bv[j]], cbuf_v.at[slot, j], sem_g).wait()
            for j in range(_L):
                @pl.when(o + j < cnt)
                def _fire_scatter():
                    pltpu.async_copy(
                        cbuf_v.at[slot, j], out_ref.at[dv[j]], sem_s)
            return 0

        nchunk = (cnt + _L - 1) // _L
        lax.fori_loop(0, nchunk, chunk_body, 0)

        fo = (nchunk - 1) * _L
        fbv = selb_v[pl.ds(fo, _L)]
        fdv = seldst_v[pl.ds(fo, _L)]
        fslot = (nchunk - 1) % 2
        for j in range(_L):
            @pl.when(fo + j < cnt)
            def _drain_final():
                pltpu.make_async_copy(
                    cbuf_v.at[fslot, j], out_ref.at[fdv[j]], sem_s).wait()


def kernel(input, index, value):
    M, D1, D2 = input.shape
    B = index.shape[0]
    rpw = (M + _NW - 1) // _NW
    rpw_pad = ((rpw + _L - 1) // _L) * _L
    cap = rpw_pad + _K

    mesh = plsc.VectorSubcoreMesh(core_axis_name="c", subcore_axis_name="s")
    sc_call = pl.kernel(
        functools.partial(_sc_scatter_body, M, B, rpw, rpw_pad),
        out_type=jax.ShapeDtypeStruct((M, D1, D2), input.dtype),
        mesh=mesh,
        compiler_params=pltpu.CompilerParams(needs_layout_passes=False),
        scratch_types=[
            pltpu.VMEM((B,), jnp.int32),
            pltpu.VMEM((rpw_pad,), jnp.int32),
            pltpu.VMEM((cap,), jnp.int32),
            pltpu.VMEM((cap,), jnp.int32),
            pltpu.VMEM((2, _CH, D1, D2), jnp.float32),
            pltpu.SemaphoreType.DMA,
            pltpu.SemaphoreType.DMA,
            pltpu.SemaphoreType.DMA,
        ],
    )

    return sc_call(index, value, input)


# speedup vs baseline: 1.0215x; 1.0215x over previous
import functools

import jax
import jax.numpy as jnp
from jax import lax
from jax.experimental import pallas as pl
from jax.experimental.pallas import tpu as pltpu
from jax.experimental.pallas import tpu_sc as plsc

_NC = 2
_NS = 16
_NW = _NC * _NS
_L = 16
_K = 32
_CH = 12


def _sc_scatter_body(M, B, rpw, rpw_pad, idx_hbm, val_hbm, in_hbm, out_ref,
                     idx_v, lastpos_v, selb_v, seldst_v, buf_v, sbuf_sh,
                     sem_c, sem_g, sem_s):
    sid = lax.axis_index("s")
    wid = lax.axis_index("s") * _NC + lax.axis_index("c")
    cbuf_v = sbuf_sh.at[sid]
    base = wid * rpw
    span_main = M - (_NW - 1) * rpw
    tail = span_main % _L

    pltpu.sync_copy(idx_hbm, idx_v)

    minus1 = jnp.full((_L,), -1, jnp.int32)

    def init_body(i, _):
        lastpos_v[pl.ds(i * _L, _L)] = minus1
        return 0

    lax.fori_loop(0, rpw_pad // _L, init_body, 0, unroll=4)

    iota = lax.iota(jnp.int32, _L)

    def scan_body(i, _):
        v = idx_v[pl.ds(i * _L, _L)]
        owned = (v >= base) & (v < base + rpw)
        _, lastmask = plsc.scan_count(v, owned)
        keep = lastmask & owned
        bvec = iota + i * _L
        plsc.store_scatter(lastpos_v, [v - base], bvec, mask=keep)
        return 0

    lax.fori_loop(0, B // _L, scan_body, 0, unroll=4)

    def compact_body(i, off):
        lp = lastpos_v[pl.ds(i * _L, _L)]
        m = lp >= 0
        plsc.store_compressed(selb_v.at[pl.ds(off, _L)], lp, mask=m)
        plsc.store_compressed(
            seldst_v.at[pl.ds(off, _L)], iota + (base + i * _L), mask=m)
        return off + jnp.sum(m.astype(jnp.int32))

    cnt = lax.fori_loop(0, rpw_pad // _L, compact_body, 0, unroll=4)

    span = jnp.where(wid == _NW - 1, span_main, rpw)
    nfull = span // _CH
    ntail = span - nfull * _CH

    def _cgather(c, slot):
        return pltpu.make_async_copy(
            in_hbm.at[pl.ds(base + c * _CH, _CH)], cbuf_v.at[slot], sem_c)

    def _cscatter(c, slot):
        return pltpu.make_async_copy(
            cbuf_v.at[slot], out_ref.at[pl.ds(base + c * _CH, _CH)], sem_s)

    _cgather(0, 0).start()

    def copy_body(c, _):
        slot = c % 2

        @pl.when(c > 0)
        def _wait_prev_store():
            _cscatter(c - 1, 1 - slot).wait()

        @pl.when(c + 1 < nfull)
        def _fire_next_load():
            _cgather(c + 1, 1 - slot).start()

        _cgather(c, slot).wait()
        _cscatter(c, slot).start()
        return 0

    lax.fori_loop(0, nfull, copy_body, 0)
    _cscatter(nfull - 1, (nfull - 1) % 2).wait()

    toff = base + nfull * _CH
    for j in range(_CH - 1):
        @pl.when(j < ntail)
        def _tg():
            pltpu.async_copy(in_hbm.at[toff + j], cbuf_v.at[0, j], sem_c)
    for j in range(_CH - 1):
        @pl.when(j < ntail)
        def _tgw():
            pltpu.make_async_copy(
                in_hbm.at[toff + j], cbuf_v.at[0, j], sem_c).wait()
    for j in range(_CH - 1):
        @pl.when(j < ntail)
        def _ts():
            pltpu.async_copy(cbuf_v.at[0, j], out_ref.at[toff + j], sem_s)
    for j in range(_CH - 1):
        @pl.when(j < ntail)
        def _tsw():
            pltpu.make_async_copy(
                cbuf_v.at[0, j], out_ref.at[toff + j], sem_s).wait()

    @pl.when(cnt > 0)
    def _move():
        def chunk_body(c, _):
            o = c * _L
            bv = selb_v[pl.ds(o, _L)]
            dv = seldst_v[pl.ds(o, _L)]
            slot = c % 2
            for j in range(_L):
                @pl.when(o + j < cnt)
                def _fire_gather():
                    pltpu.async_copy(
                        val_hbm.at[bv[j]], buf_v.at[slot, j], sem_g)

            @pl.when(c > 0)
            def _drain_prev_scatters():
                po = (c - 1) * _L
                pbv = selb_v[pl.ds(po, _L)]
                pdv = seldst_v[pl.ds(po, _L)]
                for j in range(_L):
                    @pl.when(po + j < cnt)
                    def _drain_s():
                        pltpu.make_async_copy(
                            buf_v.at[1 - slot, j], out_ref.at[pdv[j]],
                            sem_s).wait()

            for j in range(_L):
                @pl.when(o + j < cnt)
                def _drain_g():
                    pltpu.make_async_copy(
                        val_hbm.at[bv[j]], buf_v.at[slot, j], sem_g).wait()
            for j in range(_L):
                @pl.when(o + j < cnt)
                def _fire_scatter():
                    pltpu.async_copy(
                        buf_v.at[slot, j], out_ref.at[dv[j]], sem_s)
            return 0

        nchunk = (cnt + _L - 1) // _L
        lax.fori_loop(0, nchunk, chunk_body, 0)

        fo = (nchunk - 1) * _L
        fbv = selb_v[pl.ds(fo, _L)]
        fdv = seldst_v[pl.ds(fo, _L)]
        fslot = (nchunk - 1) % 2
        for j in range(_L):
            @pl.when(fo + j < cnt)
            def _drain_final():
                pltpu.make_async_copy(
                    buf_v.at[fslot, j], out_ref.at[fdv[j]], sem_s).wait()


def kernel(input, index, value):
    M, D1, D2 = input.shape
    B = index.shape[0]
    rpw = (M + _NW - 1) // _NW
    rpw_pad = ((rpw + _L - 1) // _L) * _L
    cap = rpw_pad + _K

    mesh = plsc.VectorSubcoreMesh(core_axis_name="c", subcore_axis_name="s")
    sc_call = pl.kernel(
        functools.partial(_sc_scatter_body, M, B, rpw, rpw_pad),
        out_type=jax.ShapeDtypeStruct((M, D1, D2), input.dtype),
        mesh=mesh,
        compiler_params=pltpu.CompilerParams(needs_layout_passes=False),
        scratch_types=[
            pltpu.VMEM((B,), jnp.int32),
            pltpu.VMEM((rpw_pad,), jnp.int32),
            pltpu.VMEM((cap,), jnp.int32),
            pltpu.VMEM((cap,), jnp.int32),
            pltpu.VMEM((2, _L, D1, D2), jnp.float32),
            pltpu.VMEM_SHARED((_NS, 2, _CH, D1, D2), jnp.float32),
            pltpu.SemaphoreType.DMA,
            pltpu.SemaphoreType.DMA,
            pltpu.SemaphoreType.DMA,
        ],
    )

    return sc_call(index, value, input)


# speedup vs baseline: 1.3968x; 1.3673x over previous
import functools

import jax
import jax.numpy as jnp
from jax import lax
from jax.experimental import pallas as pl
from jax.experimental.pallas import tpu as pltpu
from jax.experimental.pallas import tpu_sc as plsc

_NC = 2
_NS = 16
_NW = _NC * _NS
_L = 16
_K = 32


def _sc_scatter_body(M, B, rpw, rpw_pad, idx_hbm, val_hbm, out_ref,
                     idx_v, lastpos_v, selb_v, seldst_v, buf_v, sem_g, sem_s):
    wid = lax.axis_index("s") * _NC + lax.axis_index("c")
    base = wid * rpw

    pltpu.sync_copy(idx_hbm, idx_v)

    minus1 = jnp.full((_L,), -1, jnp.int32)

    def init_body(i, _):
        lastpos_v[pl.ds(i * _L, _L)] = minus1
        return 0

    lax.fori_loop(0, rpw_pad // _L, init_body, 0, unroll=4)

    iota = lax.iota(jnp.int32, _L)

    def scan_body(i, _):
        v = idx_v[pl.ds(i * _L, _L)]
        owned = (v >= base) & (v < base + rpw)
        _, lastmask = plsc.scan_count(v, owned)
        keep = lastmask & owned
        bvec = iota + i * _L
        plsc.store_scatter(lastpos_v, [v - base], bvec, mask=keep)
        return 0

    lax.fori_loop(0, B // _L, scan_body, 0, unroll=4)

    def compact_body(i, off):
        lp = lastpos_v[pl.ds(i * _L, _L)]
        m = lp >= 0
        plsc.store_compressed(selb_v.at[pl.ds(off, _L)], lp, mask=m)
        plsc.store_compressed(
            seldst_v.at[pl.ds(off, _L)], iota + (base + i * _L), mask=m)
        return off + jnp.sum(m.astype(jnp.int32))

    cnt = lax.fori_loop(0, rpw_pad // _L, compact_body, 0, unroll=4)

    @pl.when(cnt > 0)
    def _move():
        def chunk_body(c, _):
            o = c * _L
            bv = selb_v[pl.ds(o, _L)]
            dv = seldst_v[pl.ds(o, _L)]
            slot = c % 2
            for j in range(_L):
                @pl.when(o + j < cnt)
                def _fire_gather():
                    pltpu.async_copy(
                        val_hbm.at[bv[j]], buf_v.at[slot, j], sem_g)

            @pl.when(c > 0)
            def _drain_prev_scatters():
                po = (c - 1) * _L
                pbv = selb_v[pl.ds(po, _L)]
                pdv = seldst_v[pl.ds(po, _L)]
                for j in range(_L):
                    @pl.when(po + j < cnt)
                    def _drain_s():
                        pltpu.make_async_copy(
                            buf_v.at[1 - slot, j], out_ref.at[pdv[j]],
                            sem_s).wait()

            for j in range(_L):
                @pl.when(o + j < cnt)
                def _drain_g():
                    pltpu.make_async_copy(
                        val_hbm.at[bv[j]], buf_v.at[slot, j], sem_g).wait()
            for j in range(_L):
                @pl.when(o + j < cnt)
                def _fire_scatter():
                    pltpu.async_copy(
                        buf_v.at[slot, j], out_ref.at[dv[j]], sem_s)
            return 0

        nchunk = (cnt + _L - 1) // _L
        lax.fori_loop(0, nchunk, chunk_body, 0)

        fo = (nchunk - 1) * _L
        fbv = selb_v[pl.ds(fo, _L)]
        fdv = seldst_v[pl.ds(fo, _L)]
        fslot = (nchunk - 1) % 2
        for j in range(_L):
            @pl.when(fo + j < cnt)
            def _drain_final():
                pltpu.make_async_copy(
                    buf_v.at[fslot, j], out_ref.at[fdv[j]], sem_s).wait()


def kernel(input, index, value):
    M, D1, D2 = input.shape
    B = index.shape[0]
    rpw = (M + _NW - 1) // _NW
    rpw_pad = ((rpw + _L - 1) // _L) * _L
    cap = rpw_pad + _K

    mesh = plsc.VectorSubcoreMesh(core_axis_name="c", subcore_axis_name="s")
    sc_call = pl.kernel(
        functools.partial(_sc_scatter_body, M, B, rpw, rpw_pad),
        out_type=(),
        mesh=mesh,
        compiler_params=pltpu.CompilerParams(needs_layout_passes=False),
        scratch_types=[
            pltpu.VMEM((B,), jnp.int32),
            pltpu.VMEM((rpw_pad,), jnp.int32),
            pltpu.VMEM((cap,), jnp.int32),
            pltpu.VMEM((cap,), jnp.int32),
            pltpu.VMEM((2, _L, D1, D2), jnp.float32),
            pltpu.SemaphoreType.DMA,
            pltpu.SemaphoreType.DMA,
        ],
    )

    out_ref = jax.new_ref(input)
    sc_call(index, value, out_ref)
    return out_ref[...]


# speedup vs baseline: 2.2395x; 1.6034x over previous
import functools

import jax
import jax.numpy as jnp
from jax import lax
from jax.experimental import pallas as pl
from jax.experimental.pallas import tpu as pltpu
from jax.experimental.pallas import tpu_sc as plsc

_NC = 2
_NS = 16
_NW = _NC * _NS
_L = 16


def _sc_scatter_body(M, B, D, rpw, rpw_pad, idx_hbm, val_hbm, out_ref,
                     idx_v, lastpos_v, selb_v, seldst_v, buf_v, sem_g, sem_s):
    wid = lax.axis_index("s") * _NC + lax.axis_index("c")
    base = wid * rpw

    pltpu.sync_copy(idx_hbm, idx_v)

    minus1 = jnp.full((_L,), -1, jnp.int32)

    def init_body(i, _):
        lastpos_v[pl.ds(i * _L, _L)] = minus1
        return 0

    lax.fori_loop(0, rpw_pad // _L, init_body, 0, unroll=4)

    iota = lax.iota(jnp.int32, _L)

    def scan_body(i, _):
        v = idx_v[pl.ds(i * _L, _L)]
        owned = (v >= base) & (v < base + rpw)
        _, lastmask = plsc.scan_count(v, owned)
        keep = lastmask & owned
        bvec = iota + i * _L
        plsc.store_scatter(lastpos_v, [v - base], bvec, mask=keep)
        return 0

    lax.fori_loop(0, B // _L, scan_body, 0, unroll=4)

    def compact_body(i, off):
        lp = lastpos_v[pl.ds(i * _L, _L)]
        m = lp >= 0
        plsc.store_compressed(selb_v.at[pl.ds(off, _L)], lp, mask=m)
        plsc.store_compressed(
            seldst_v.at[pl.ds(off, _L)], iota + (base + i * _L), mask=m)
        return off + jnp.sum(m.astype(jnp.int32))

    cnt = lax.fori_loop(0, rpw_pad // _L, compact_body, 0, unroll=4)

    @pl.when(cnt > 0)
    def _move():
        def chunk_body(c, _):
            o = c * _L
            bv = selb_v[pl.ds(o, _L)]
            dv = seldst_v[pl.ds(o, _L)]
            slot = c % 2
            for j in range(_L):
                @pl.when(o + j < cnt)
                def _fire_gather():
                    pltpu.async_copy(
                        val_hbm.at[bv[j]], buf_v.at[slot, j], sem_g)

            @pl.when(c > 0)
            def _drain_prev_scatters():
                po = (c - 1) * _L
                pdv = seldst_v[pl.ds(po, _L)]
                for j in range(_L):
                    @pl.when(po + j < cnt)
                    def _drain_s():
                        pltpu.make_async_copy(
                            buf_v.at[1 - slot, j], out_ref.at[pdv[j]],
                            sem_s).wait()

            for j in range(_L):
                @pl.when(o + j < cnt)
                def _drain_g():
                    pltpu.make_async_copy(
                        val_hbm.at[bv[j]], buf_v.at[slot, j], sem_g).wait()
            for j in range(_L):
                @pl.when(o + j < cnt)
                def _fire_scatter():
                    pltpu.async_copy(
                        buf_v.at[slot, j], out_ref.at[dv[j]], sem_s)
            return 0

        nchunk = (cnt + _L - 1) // _L
        lax.fori_loop(0, nchunk, chunk_body, 0)

        fo = (nchunk - 1) * _L
        fdv = seldst_v[pl.ds(fo, _L)]
        fslot = (nchunk - 1) % 2
        for j in range(_L):
            @pl.when(fo + j < cnt)
            def _drain_final():
                pltpu.make_async_copy(
                    buf_v.at[fslot, j], out_ref.at[fdv[j]], sem_s).wait()


def kernel(input, index, value):
    M, D1, D2 = input.shape
    B = index.shape[0]
    D = D1 * D2
    rpw = (M + _NW - 1) // _NW
    rpw_pad = ((rpw + _L - 1) // _L) * _L
    cap = rpw_pad + _L

    mesh = plsc.VectorSubcoreMesh(core_axis_name="c", subcore_axis_name="s")
    sc_call = pl.kernel(
        functools.partial(_sc_scatter_body, M, B, D, rpw, rpw_pad),
        out_type=(),
        mesh=mesh,
        compiler_params=pltpu.CompilerParams(needs_layout_passes=False),
        scratch_types=[
            pltpu.VMEM((B,), jnp.int32),
            pltpu.VMEM((rpw_pad,), jnp.int32),
            pltpu.VMEM((cap,), jnp.int32),
            pltpu.VMEM((cap,), jnp.int32),
            pltpu.VMEM((2, _L, D), jnp.float32),
            pltpu.SemaphoreType.DMA,
            pltpu.SemaphoreType.DMA,
        ],
    )

    out_ref = jax.new_ref(input.reshape(M, D))
    sc_call(index, value.reshape(B, D), out_ref)
    return out_ref[...].reshape(M, D1, D2)


# speedup vs baseline: 2.2857x; 1.0206x over previous
import functools

import jax
import jax.numpy as jnp
from jax import lax
from jax.experimental import pallas as pl
from jax.experimental.pallas import tpu as pltpu
from jax.experimental.pallas import tpu_sc as plsc

_NC = 2
_NS = 16
_NW = _NC * _NS
_L = 16


def _sc_scatter_body(M, B, D, rpw, rpw_pad, idx_hbm, val_hbm, out_ref,
                     idx_v, lastpos_v, selb_v, seldst_v, buf_v, sem_g, sem_s):
    wid = lax.axis_index("s") * _NC + lax.axis_index("c")
    base = wid * rpw

    pltpu.sync_copy(idx_hbm, idx_v)

    minus1 = jnp.full((_L,), -1, jnp.int32)

    def init_body(i, _):
        lastpos_v[pl.ds(i * _L, _L)] = minus1
        return 0

    lax.fori_loop(0, rpw_pad // _L, init_body, 0, unroll=4)

    iota = lax.iota(jnp.int32, _L)

    def scan_body(i, _):
        v = idx_v[pl.ds(i * _L, _L)]
        owned = (v >= base) & (v < base + rpw)
        _, lastmask = plsc.scan_count(v, owned)
        keep = lastmask & owned
        bvec = iota + i * _L
        plsc.store_scatter(lastpos_v, [v - base], bvec, mask=keep)
        return 0

    lax.fori_loop(0, B // _L, scan_body, 0, unroll=4)

    def compact_body(i, off):
        lp = lastpos_v[pl.ds(i * _L, _L)]
        m = lp >= 0
        plsc.store_compressed(selb_v.at[pl.ds(off, _L)], lp, mask=m)
        plsc.store_compressed(
            seldst_v.at[pl.ds(off, _L)], iota + (base + i * _L), mask=m)
        return off + jnp.sum(m.astype(jnp.int32))

    cnt = lax.fori_loop(0, rpw_pad // _L, compact_body, 0, unroll=4)

    @pl.when(cnt > 0)
    def _move():
        vbase = (cnt // _L) * _L

        @pl.when(vbase < cnt)
        def _pad_tail():
            lastv = selb_v[pl.ds(cnt - 1, _L)]
            lastd = seldst_v[pl.ds(cnt - 1, _L)]
            b_pad = jnp.full((_L,), lastv[0], jnp.int32)
            d_pad = jnp.full((_L,), lastd[0], jnp.int32)
            fill = iota >= (cnt - vbase)
            bv0 = selb_v[pl.ds(vbase, _L)]
            dv0 = seldst_v[pl.ds(vbase, _L)]
            selb_v[pl.ds(vbase, _L)] = jnp.where(fill, b_pad, bv0)
            seldst_v[pl.ds(vbase, _L)] = jnp.where(fill, d_pad, dv0)

        nchunk = (cnt + _L - 1) // _L

        def _ig(c, slot):
            return pltpu.make_async_copy(
                val_hbm.at[selb_v.at[pl.ds(c * _L, _L)]],
                buf_v.at[slot], sem_g)

        def _is(c, slot):
            return pltpu.make_async_copy(
                buf_v.at[slot],
                out_ref.at[seldst_v.at[pl.ds(c * _L, _L)]], sem_s)

        _ig(0, 0).start()

        def chunk_body(c, _):
            slot = c % 2

            @pl.when(c > 0)
            def _drain_prev_scatter():
                _is(c - 1, 1 - slot).wait()

            @pl.when(c + 1 < nchunk)
            def _fire_next_gather():
                _ig(c + 1, 1 - slot).start()

            _ig(c, slot).wait()
            _is(c, slot).start()
            return 0

        lax.fori_loop(0, nchunk, chunk_body, 0)
        _is(nchunk - 1, (nchunk - 1) % 2).wait()


def kernel(input, index, value):
    M, D1, D2 = input.shape
    B = index.shape[0]
    D = D1 * D2
    rpw = (M + _NW - 1) // _NW
    rpw_pad = ((rpw + _L - 1) // _L) * _L
    cap = rpw_pad + _L

    mesh = plsc.VectorSubcoreMesh(core_axis_name="c", subcore_axis_name="s")
    sc_call = pl.kernel(
        functools.partial(_sc_scatter_body, M, B, D, rpw, rpw_pad),
        out_type=(),
        mesh=mesh,
        compiler_params=pltpu.CompilerParams(needs_layout_passes=False),
        scratch_types=[
            pltpu.VMEM((B,), jnp.int32),
            pltpu.VMEM((rpw_pad,), jnp.int32),
            pltpu.VMEM((cap,), jnp.int32),
            pltpu.VMEM((cap,), jnp.int32),
            pltpu.VMEM((2, _L, D), jnp.float32),
            pltpu.SemaphoreType.DMA,
            pltpu.SemaphoreType.DMA,
        ],
    )

    out_ref = jax.new_ref(input.reshape(M, D))
    sc_call(index, value.reshape(B, D), out_ref)
    return out_ref[...].reshape(M, D1, D2)
